# Initial kernel scaffold; baseline (speedup 1.0000x reference)
import jax
import jax.numpy as jnp
from jax.experimental import pallas as pl


def kernel(x, edge_index, gin0_w1, gin0_b1, gin0_w2, gin0_b2, gin1_w1, gin1_b1, gin1_w2, gin1_b2, gin2_w1, gin2_b1, gin2_w2, gin2_b2, proj_w, proj_b, act_alpha, proj_alpha, enc_bn_scale, enc_bn_shift, proj_bn_scale, proj_bn_shift):
    raise NotImplementedError("write your pallas kernel here")


# speedup vs baseline: 7.8239x; 7.8239x over previous
import functools

import jax
import jax.numpy as jnp
from jax.experimental import pallas as pl
from jax.experimental.pallas import tpu as pltpu

_N = 16384
_H = 256
_TM = 1024
_TK = 4096


def _gin_mid_kernel(a_ref, z_ref, zd_ref, w1_ref, b1_ref, w2_ref, b2_ref,
                    alpha_ref, o_ref, acc_ref):
    k = pl.program_id(1)

    @pl.when(k == 0)
    def _():
        acc_ref[...] = zd_ref[...].astype(jnp.float32)

    acc_ref[...] += jnp.dot(a_ref[...], z_ref[...],
                            preferred_element_type=jnp.float32)

    @pl.when(k == pl.num_programs(1) - 1)
    def _():
        h = jnp.dot(acc_ref[...], w1_ref[...],
                    preferred_element_type=jnp.float32) + b1_ref[...]
        h = jnp.maximum(h, 0.0)
        y = jnp.dot(h, w2_ref[...],
                    preferred_element_type=jnp.float32) + b2_ref[...]
        alpha = alpha_ref[0]
        y = jnp.where(y >= 0.0, y, alpha * y)
        o_ref[...] = y.astype(o_ref.dtype)


def _gin_last_kernel(a_ref, z_ref, zd_ref, w1_ref, b1_ref, w2_ref, b2_ref,
                     scale_ref, shift_ref, wp_ref, bp_ref,
                     alpha_ref, palpha_ref, z_out_ref, p_out_ref, acc_ref):
    k = pl.program_id(1)

    @pl.when(k == 0)
    def _():
        acc_ref[...] = zd_ref[...].astype(jnp.float32)

    acc_ref[...] += jnp.dot(a_ref[...], z_ref[...],
                            preferred_element_type=jnp.float32)

    @pl.when(k == pl.num_programs(1) - 1)
    def _():
        h = jnp.dot(acc_ref[...], w1_ref[...],
                    preferred_element_type=jnp.float32) + b1_ref[...]
        h = jnp.maximum(h, 0.0)
        y = jnp.dot(h, w2_ref[...],
                    preferred_element_type=jnp.float32) + b2_ref[...]
        alpha = alpha_ref[0]
        y = jnp.where(y >= 0.0, y, alpha * y)
        z = y * scale_ref[...] + shift_ref[...]
        z_out_ref[...] = z
        p = jnp.dot(z, wp_ref[...],
                    preferred_element_type=jnp.float32) + bp_ref[...]
        palpha = palpha_ref[0]
        p_out_ref[...] = jnp.where(p >= 0.0, p, palpha * p)


def _row(v):
    return v.reshape(1, -1).astype(jnp.float32)


def _gin_mid(a_hat, z, w1, b1, w2, b2, alpha):
    grid = (_N // _TM, _N // _TK)
    return pl.pallas_call(
        _gin_mid_kernel,
        out_shape=jax.ShapeDtypeStruct((_N, _H), jnp.bfloat16),
        grid=grid,
        in_specs=[
            pl.BlockSpec((_TM, _TK), lambda i, k: (i, k)),
            pl.BlockSpec((_TK, _H), lambda i, k: (k, 0)),
            pl.BlockSpec((_TM, _H), lambda i, k: (i, 0)),
            pl.BlockSpec((_H, _H), lambda i, k: (0, 0)),
            pl.BlockSpec((1, _H), lambda i, k: (0, 0)),
            pl.BlockSpec((_H, _H), lambda i, k: (0, 0)),
            pl.BlockSpec((1, _H), lambda i, k: (0, 0)),
            pl.BlockSpec(memory_space=pltpu.MemorySpace.SMEM),
        ],
        out_specs=pl.BlockSpec((_TM, _H), lambda i, k: (i, 0)),
        scratch_shapes=[pltpu.VMEM((_TM, _H), jnp.float32)],
        compiler_params=pltpu.CompilerParams(
            dimension_semantics=("parallel", "arbitrary")),
        cost_estimate=pl.CostEstimate(
            flops=2 * _N * _N * _H + 4 * _N * _H * _H,
            transcendentals=0,
            bytes_accessed=2 * _N * _N + 2 * 2 * _N * _H + 2 * _N * _H
                           + 8 * _H * _H),
    )(a_hat, z, z, w1, b1, w2, b2, alpha)


def _gin_last(a_hat, z, w1, b1, w2, b2, scale, shift, wp, bp, alpha, palpha):
    grid = (_N // _TM, _N // _TK)
    return pl.pallas_call(
        _gin_last_kernel,
        out_shape=(jax.ShapeDtypeStruct((_N, _H), jnp.float32),
                   jax.ShapeDtypeStruct((_N, _H), jnp.float32)),
        grid=grid,
        in_specs=[
            pl.BlockSpec((_TM, _TK), lambda i, k: (i, k)),
            pl.BlockSpec((_TK, _H), lambda i, k: (k, 0)),
            pl.BlockSpec((_TM, _H), lambda i, k: (i, 0)),
            pl.BlockSpec((_H, _H), lambda i, k: (0, 0)),
            pl.BlockSpec((1, _H), lambda i, k: (0, 0)),
            pl.BlockSpec((_H, _H), lambda i, k: (0, 0)),
            pl.BlockSpec((1, _H), lambda i, k: (0, 0)),
            pl.BlockSpec((1, _H), lambda i, k: (0, 0)),
            pl.BlockSpec((1, _H), lambda i, k: (0, 0)),
            pl.BlockSpec((_H, _H), lambda i, k: (0, 0)),
            pl.BlockSpec((1, _H), lambda i, k: (0, 0)),
            pl.BlockSpec(memory_space=pltpu.MemorySpace.SMEM),
            pl.BlockSpec(memory_space=pltpu.MemorySpace.SMEM),
        ],
        out_specs=(pl.BlockSpec((_TM, _H), lambda i, k: (i, 0)),
                   pl.BlockSpec((_TM, _H), lambda i, k: (i, 0))),
        scratch_shapes=[pltpu.VMEM((_TM, _H), jnp.float32)],
        compiler_params=pltpu.CompilerParams(
            dimension_semantics=("parallel", "arbitrary")),
        cost_estimate=pl.CostEstimate(
            flops=2 * _N * _N * _H + 6 * _N * _H * _H,
            transcendentals=0,
            bytes_accessed=2 * _N * _N + 2 * 2 * _N * _H + 8 * _N * _H
                           + 12 * _H * _H),
    )(a_hat, z, z, w1, b1, w2, b2, scale, shift, wp, bp, alpha, palpha)


def kernel(x, edge_index,
           gin0_w1, gin0_b1, gin0_w2, gin0_b2,
           gin1_w1, gin1_b1, gin1_w2, gin1_b2,
           gin2_w1, gin2_b1, gin2_w2, gin2_b2,
           proj_w, proj_b, act_alpha, proj_alpha,
           enc_bn_scale, enc_bn_shift, proj_bn_scale, proj_bn_shift):
    src, dst = edge_index[0], edge_index[1]
    a_hat = jnp.zeros((_N, _N), jnp.bfloat16).at[dst, src].add(
        jnp.ones(src.shape, jnp.bfloat16))

    alpha = jnp.asarray(act_alpha, jnp.float32).reshape(1)
    palpha = jnp.asarray(proj_alpha, jnp.float32).reshape(1)

    wp = proj_w * proj_bn_scale[None, :]
    bp = proj_b * proj_bn_scale + proj_bn_shift

    z = x.astype(jnp.bfloat16)
    z = _gin_mid(a_hat, z, gin0_w1, _row(gin0_b1), gin0_w2, _row(gin0_b2),
                 alpha)
    z = _gin_mid(a_hat, z, gin1_w1, _row(gin1_b1), gin1_w2, _row(gin1_b2),
                 alpha)
    z3, p = _gin_last(a_hat, z, gin2_w1, _row(gin2_b1), gin2_w2,
                      _row(gin2_b2), _row(enc_bn_scale), _row(enc_bn_shift),
                      wp, _row(bp), alpha, palpha)
    return z3, p


# speedup vs baseline: 7.8952x; 1.0091x over previous
import functools

import jax
import jax.numpy as jnp
from jax.experimental import pallas as pl
from jax.experimental.pallas import tpu as pltpu

_N = 16384
_H = 256
_TM = 1024
_TK = 4096


def _gin_mid_kernel(a_ref, z_ref, zd_ref, w1_ref, b1_ref, w2_ref, b2_ref,
                    alpha_ref, o_ref, acc_ref):
    k = pl.program_id(1)

    @pl.when(k == 0)
    def _():
        acc_ref[...] = zd_ref[...].astype(jnp.float32)

    acc_ref[...] += jnp.dot(a_ref[...], z_ref[...],
                            preferred_element_type=jnp.float32)

    @pl.when(k == pl.num_programs(1) - 1)
    def _():
        h = jnp.dot(acc_ref[...], w1_ref[...],
                    preferred_element_type=jnp.float32) + b1_ref[...]
        h = jnp.maximum(h, 0.0)
        y = jnp.dot(h, w2_ref[...],
                    preferred_element_type=jnp.float32) + b2_ref[...]
        alpha = alpha_ref[0]
        y = jnp.where(y >= 0.0, y, alpha * y)
        o_ref[...] = y.astype(o_ref.dtype)


def _gin_last_kernel(a_ref, z_ref, zd_ref, w1_ref, b1_ref, w2_ref, b2_ref,
                     scale_ref, shift_ref, wp_ref, bp_ref,
                     alpha_ref, palpha_ref, z_out_ref, p_out_ref, acc_ref):
    k = pl.program_id(1)

    @pl.when(k == 0)
    def _():
        acc_ref[...] = zd_ref[...].astype(jnp.float32)

    acc_ref[...] += jnp.dot(a_ref[...], z_ref[...],
                            preferred_element_type=jnp.float32)

    @pl.when(k == pl.num_programs(1) - 1)
    def _():
        h = jnp.dot(acc_ref[...], w1_ref[...],
                    preferred_element_type=jnp.float32) + b1_ref[...]
        h = jnp.maximum(h, 0.0)
        y = jnp.dot(h, w2_ref[...],
                    preferred_element_type=jnp.float32) + b2_ref[...]
        alpha = alpha_ref[0]
        y = jnp.where(y >= 0.0, y, alpha * y)
        z = y * scale_ref[...] + shift_ref[...]
        z_out_ref[...] = z
        p = jnp.dot(z, wp_ref[...],
                    preferred_element_type=jnp.float32) + bp_ref[...]
        palpha = palpha_ref[0]
        p_out_ref[...] = jnp.where(p >= 0.0, p, palpha * p)


def _row(v):
    return v.reshape(1, -1).astype(jnp.float32)


def _gin_mid(a_hat, z, w1, b1, w2, b2, alpha):
    grid = (_N // _TM, _N // _TK)
    return pl.pallas_call(
        _gin_mid_kernel,
        out_shape=jax.ShapeDtypeStruct((_N, _H), jnp.bfloat16),
        grid=grid,
        in_specs=[
            pl.BlockSpec((_TM, _TK), lambda i, k: (i, k)),
            pl.BlockSpec((_TK, _H), lambda i, k: (k, 0)),
            pl.BlockSpec((_TM, _H), lambda i, k: (i, 0)),
            pl.BlockSpec((_H, _H), lambda i, k: (0, 0)),
            pl.BlockSpec((1, _H), lambda i, k: (0, 0)),
            pl.BlockSpec((_H, _H), lambda i, k: (0, 0)),
            pl.BlockSpec((1, _H), lambda i, k: (0, 0)),
            pl.BlockSpec(memory_space=pltpu.MemorySpace.SMEM),
        ],
        out_specs=pl.BlockSpec((_TM, _H), lambda i, k: (i, 0)),
        scratch_shapes=[pltpu.VMEM((_TM, _H), jnp.float32)],
        compiler_params=pltpu.CompilerParams(
            dimension_semantics=("parallel", "arbitrary")),
        cost_estimate=pl.CostEstimate(
            flops=2 * _N * _N * _H + 4 * _N * _H * _H,
            transcendentals=0,
            bytes_accessed=2 * _N * _N + 2 * 2 * _N * _H + 2 * _N * _H
                           + 8 * _H * _H),
    )(a_hat, z, z, w1, b1, w2, b2, alpha)


def _gin_last(a_hat, z, w1, b1, w2, b2, scale, shift, wp, bp, alpha, palpha):
    grid = (_N // _TM, _N // _TK)
    return pl.pallas_call(
        _gin_last_kernel,
        out_shape=(jax.ShapeDtypeStruct((_N, _H), jnp.float32),
                   jax.ShapeDtypeStruct((_N, _H), jnp.float32)),
        grid=grid,
        in_specs=[
            pl.BlockSpec((_TM, _TK), lambda i, k: (i, k)),
            pl.BlockSpec((_TK, _H), lambda i, k: (k, 0)),
            pl.BlockSpec((_TM, _H), lambda i, k: (i, 0)),
            pl.BlockSpec((_H, _H), lambda i, k: (0, 0)),
            pl.BlockSpec((1, _H), lambda i, k: (0, 0)),
            pl.BlockSpec((_H, _H), lambda i, k: (0, 0)),
            pl.BlockSpec((1, _H), lambda i, k: (0, 0)),
            pl.BlockSpec((1, _H), lambda i, k: (0, 0)),
            pl.BlockSpec((1, _H), lambda i, k: (0, 0)),
            pl.BlockSpec((_H, _H), lambda i, k: (0, 0)),
            pl.BlockSpec((1, _H), lambda i, k: (0, 0)),
            pl.BlockSpec(memory_space=pltpu.MemorySpace.SMEM),
            pl.BlockSpec(memory_space=pltpu.MemorySpace.SMEM),
        ],
        out_specs=(pl.BlockSpec((_TM, _H), lambda i, k: (i, 0)),
                   pl.BlockSpec((_TM, _H), lambda i, k: (i, 0))),
        scratch_shapes=[pltpu.VMEM((_TM, _H), jnp.float32)],
        compiler_params=pltpu.CompilerParams(
            dimension_semantics=("parallel", "arbitrary")),
        cost_estimate=pl.CostEstimate(
            flops=2 * _N * _N * _H + 6 * _N * _H * _H,
            transcendentals=0,
            bytes_accessed=2 * _N * _N + 2 * 2 * _N * _H + 8 * _N * _H
                           + 12 * _H * _H),
    )(a_hat, z, z, w1, b1, w2, b2, scale, shift, wp, bp, alpha, palpha)


def kernel(x, edge_index,
           gin0_w1, gin0_b1, gin0_w2, gin0_b2,
           gin1_w1, gin1_b1, gin1_w2, gin1_b2,
           gin2_w1, gin2_b1, gin2_w2, gin2_b2,
           proj_w, proj_b, act_alpha, proj_alpha,
           enc_bn_scale, enc_bn_shift, proj_bn_scale, proj_bn_shift):
    src, dst = edge_index[0], edge_index[1]
    lin = dst * _N + src
    a_hat = jnp.zeros((_N * _N,), jnp.bfloat16).at[lin].add(
        jnp.ones(lin.shape, jnp.bfloat16)).reshape(_N, _N)

    alpha = jnp.asarray(act_alpha, jnp.float32).reshape(1)
    palpha = jnp.asarray(proj_alpha, jnp.float32).reshape(1)

    wp = proj_w * proj_bn_scale[None, :]
    bp = proj_b * proj_bn_scale + proj_bn_shift

    z = x.astype(jnp.bfloat16)
    z = _gin_mid(a_hat, z, gin0_w1, _row(gin0_b1), gin0_w2, _row(gin0_b2),
                 alpha)
    z = _gin_mid(a_hat, z, gin1_w1, _row(gin1_b1), gin1_w2, _row(gin1_b2),
                 alpha)
    z3, p = _gin_last(a_hat, z, gin2_w1, _row(gin2_b1), gin2_w2,
                      _row(gin2_b2), _row(enc_bn_scale), _row(enc_bn_shift),
                      wp, _row(bp), alpha, palpha)
    return z3, p
